# baseline (speedup 1.0000x reference)
import jax
import jax.numpy as jnp
from jax import lax
from jax.experimental import pallas as pl
from jax.experimental.pallas import tpu as pltpu
from jax.experimental.pallas import tpu_sc as plsc

N_NEURONS = 10000
N_EDGES = 500000
INPUT_SIZE = 512
OUTPUT_SIZE = 128
BATCH = 64
PROP_STEPS = 5

NS = 16
BLK = 256
SBE = 2048
NSB = 16
E_PAD = NS * NSB * SBE
N_PAD = 10240
CHUNK = 128
ROWCHUNKS = N_PAD // (NS * CHUNK)


def _gelu_erf(v):
    z = v * 0.7071067811865476
    az = jnp.abs(z)
    t = 1.0 / (1.0 + 0.3275911 * az)
    poly = t * (0.254829592 + t * (-0.284496736 + t * (1.421413741
           + t * (-1.453152027 + t * 1.061405429))))
    erf_abs = 1.0 - poly * jnp.exp(-az * az)
    erf = jnp.where(z < 0.0, -erf_abs, erf_abs)
    return 0.5 * v * (1.0 + erf)


def _sc_body(xt_hbm, col_hbm, row_hbm, w_hbm, zeros_hbm, out_hbm,
             s_sh, a_sh, col_s, rid_s, w_s, rows_a, rows_b,
             gsem_a, gsem_b, ssem_a, ssem_b):
    t = lax.axis_index("s")

    for k in range(ROWCHUNKS):
        pltpu.sync_copy(zeros_hbm, s_sh.at[pl.ds((t * ROWCHUNKS + k) * CHUNK,
                                                 CHUNK)])
    plsc.subcore_barrier()
    xrows = INPUT_SIZE // NS
    pltpu.sync_copy(xt_hbm.at[pl.ds(t * xrows, xrows)],
                    s_sh.at[pl.ds(t * xrows, xrows)])
    plsc.subcore_barrier()

    def col_at(k):
        return col_s.at[pl.ds(k * BLK, BLK)]

    def rid_at(k):
        return rid_s.at[pl.ds(k * BLK, BLK)]

    def issue_g(k, rows, sem):
        pltpu.async_copy(s_sh.at[col_at(k)], rows, sem)

    def wait_g(k, rows, sem):
        pltpu.make_async_copy(s_sh.at[col_at(k)], rows, sem).wait()

    def issue_s(k, rows, sem):
        pltpu.async_copy(rows, a_sh.at[rid_at(k)], sem, add=True)

    def wait_s(k, rows, sem):
        pltpu.make_async_copy(rows, a_sh.at[rid_at(k)], sem).wait()

    def scale(k, rows):
        @plsc.parallel_loop(0, BLK // 16, 1, unroll=2)
        def _(g):
            wv = w_s[pl.ds(k * BLK + g * 16, 16)]
            for j in range(16):
                e = g * 16 + j
                w = wv[j]
                for q in range(4):
                    sl = pl.ds(q * 16, 16)
                    rows[e, sl] = w * rows[e, sl]

    def step_body(step, carry):
        del step
        for k in range(ROWCHUNKS):
            pltpu.sync_copy(zeros_hbm,
                            a_sh.at[pl.ds((t * ROWCHUNKS + k) * CHUNK, CHUNK)])
        plsc.subcore_barrier()

        @pl.loop(0, NSB)
        def _(sb):
            pltpu.sync_copy(col_hbm.at[t, sb], col_s)
            pltpu.sync_copy(row_hbm.at[t, sb], rid_s)
            pltpu.sync_copy(w_hbm.at[t, sb], w_s)
            issue_g(0, rows_a, gsem_a)

            @pl.loop(0, SBE // BLK // 2)
            def _(p):
                a = 2 * p
                b = 2 * p + 1
                wait_g(a, rows_a, gsem_a)

                @pl.when(p > 0)
                def _():
                    wait_s(b - 2, rows_b, ssem_b)

                issue_g(b, rows_b, gsem_b)
                scale(a, rows_a)
                issue_s(a, rows_a, ssem_a)
                wait_g(b, rows_b, gsem_b)
                scale(b, rows_b)
                wait_s(a, rows_a, ssem_a)

                @pl.when(p < SBE // BLK // 2 - 1)
                def _():
                    issue_g(a + 2, rows_a, gsem_a)

                issue_s(b, rows_b, ssem_b)

            wait_s(SBE // BLK - 1, rows_b, ssem_b)

        plsc.subcore_barrier()

        for k in range(ROWCHUNKS):
            base = (t * ROWCHUNKS + k) * CHUNK
            pltpu.sync_copy(s_sh.at[pl.ds(base, CHUNK)],
                            rows_a.at[pl.ds(0, CHUNK)])
            pltpu.sync_copy(a_sh.at[pl.ds(base, CHUNK)],
                            rows_a.at[pl.ds(CHUNK, CHUNK)])

            @pl.loop(0, CHUNK)
            def _(r):
                for q in range(4):
                    sl = pl.ds(q * 16, 16)
                    rows_a[r, sl] = _gelu_erf(rows_a[r, sl]
                                              + rows_a[CHUNK + r, sl])

            pltpu.sync_copy(rows_a.at[pl.ds(0, CHUNK)],
                            s_sh.at[pl.ds(base, CHUNK)])
        plsc.subcore_barrier()
        return carry

    lax.fori_loop(0, PROP_STEPS, step_body, 0)

    orows = OUTPUT_SIZE // NS
    pltpu.sync_copy(s_sh.at[pl.ds(INPUT_SIZE + t * orows, orows)],
                    out_hbm.at[pl.ds(t * orows, orows)])


@jax.jit
def kernel(x, weights, edge_index):
    row = edge_index[0]
    col = edge_index[1]
    pad = E_PAD - N_EDGES
    pad_idx = (jnp.arange(pad, dtype=jnp.int32) % N_NEURONS)
    col_p = jnp.concatenate([col, pad_idx]).reshape(NS, NSB, SBE)
    row_p = jnp.concatenate([row, pad_idx]).reshape(NS, NSB, SBE)
    w_p = jnp.concatenate(
        [weights, jnp.zeros((pad,), jnp.float32)]).reshape(NS, NSB, SBE)
    xt = x.T

    mesh = plsc.VectorSubcoreMesh(core_axis_name="c", subcore_axis_name="s",
                                  num_cores=1, num_subcores=NS)
    run = pl.kernel(
        _sc_body,
        out_type=jax.ShapeDtypeStruct((OUTPUT_SIZE, BATCH), jnp.float32),
        mesh=mesh,
        compiler_params=pltpu.CompilerParams(use_tc_tiling_on_sc=False),
        scratch_types=[
            pltpu.VMEM_SHARED((N_PAD, BATCH), jnp.float32),
            pltpu.VMEM_SHARED((N_PAD, BATCH), jnp.float32),
            pltpu.VMEM((SBE,), jnp.int32),
            pltpu.VMEM((SBE,), jnp.int32),
            pltpu.VMEM((SBE,), jnp.float32),
            pltpu.VMEM((BLK, BATCH), jnp.float32),
            pltpu.VMEM((BLK, BATCH), jnp.float32),
            pltpu.SemaphoreType.DMA,
            pltpu.SemaphoreType.DMA,
            pltpu.SemaphoreType.DMA,
            pltpu.SemaphoreType.DMA,
        ],
    )
    zeros_blk = jnp.zeros((CHUNK, BATCH), jnp.float32)
    out = run(xt, col_p, row_p, w_p, zeros_blk)
    return out.T


# speedup vs baseline: 1.0277x; 1.0277x over previous
import jax
import jax.numpy as jnp
from jax import lax
from jax.experimental import pallas as pl
from jax.experimental.pallas import tpu as pltpu
from jax.experimental.pallas import tpu_sc as plsc

N_NEURONS = 10000
N_EDGES = 500000
INPUT_SIZE = 512
OUTPUT_SIZE = 128
BATCH = 64
PROP_STEPS = 5

NS = 16
BLK = 256
SBE = 2048
NSB = 16
E_PAD = NS * NSB * SBE
N_PAD = 10240
CHUNK = 128
ROWCHUNKS = N_PAD // (NS * CHUNK)


def _gelu_erf(v):
    z = v * 0.7071067811865476
    az = jnp.abs(z)
    t = 1.0 / (1.0 + 0.3275911 * az)
    poly = t * (0.254829592 + t * (-0.284496736 + t * (1.421413741
           + t * (-1.453152027 + t * 1.061405429))))
    erf_abs = 1.0 - poly * jnp.exp(-az * az)
    erf = jnp.where(z < 0.0, -erf_abs, erf_abs)
    return 0.5 * v * (1.0 + erf)


def _sc_body(xt_hbm, col_hbm, row_hbm, w_hbm, zeros_hbm, out_hbm,
             s_sh, a_sh, col_s, rid_s, w_s, rows_a, rows_b,
             gsem_a, gsem_b, ssem_a, ssem_b):
    t = lax.axis_index("s")

    for k in range(ROWCHUNKS):
        pltpu.sync_copy(zeros_hbm, s_sh.at[pl.ds((t * ROWCHUNKS + k) * CHUNK,
                                                 CHUNK)])
    plsc.subcore_barrier()
    xrows = INPUT_SIZE // NS
    pltpu.sync_copy(xt_hbm.at[pl.ds(t * xrows, xrows)],
                    s_sh.at[pl.ds(t * xrows, xrows)])
    plsc.subcore_barrier()

    def col_at(k):
        return col_s.at[pl.ds(k * BLK, BLK)]

    def rid_at(k):
        return rid_s.at[pl.ds(k * BLK, BLK)]

    def issue_g(k, rows, sem):
        pltpu.async_copy(s_sh.at[col_at(k)], rows, sem)

    def wait_g(k, rows, sem):
        pltpu.make_async_copy(s_sh.at[col_at(k)], rows, sem).wait()

    def issue_s(k, rows, sem):
        pltpu.async_copy(rows, a_sh.at[rid_at(k)], sem, add=True)

    def wait_s(k, rows, sem):
        pltpu.make_async_copy(rows, a_sh.at[rid_at(k)], sem).wait()

    def scale(k, rows):
        @plsc.parallel_loop(0, BLK // 16, 1, unroll=2)
        def _(g):
            wv = w_s[pl.ds(k * BLK + g * 16, 16)]
            for j in range(16):
                e = g * 16 + j
                w = wv[j]
                for q in range(4):
                    sl = pl.ds(q * 16, 16)
                    rows[e, sl] = w * rows[e, sl]

    def step_body(step, carry):
        del step
        for k in range(ROWCHUNKS):
            pltpu.sync_copy(zeros_hbm,
                            a_sh.at[pl.ds((t * ROWCHUNKS + k) * CHUNK, CHUNK)])
        plsc.subcore_barrier()

        @pl.loop(0, NSB)
        def _(sb):
            pltpu.sync_copy(col_hbm.at[t, sb], col_s)
            pltpu.sync_copy(row_hbm.at[t, sb], rid_s)
            pltpu.sync_copy(w_hbm.at[t, sb], w_s)
            issue_g(0, rows_a, gsem_a)

            @pl.loop(0, SBE // BLK // 2)
            def _(p):
                a = 2 * p
                b = 2 * p + 1
                wait_g(a, rows_a, gsem_a)

                @pl.when(p > 0)
                def _():
                    wait_s(b - 2, rows_b, ssem_b)

                issue_g(b, rows_b, gsem_b)
                scale(a, rows_a)
                issue_s(a, rows_a, ssem_a)
                wait_g(b, rows_b, gsem_b)
                scale(b, rows_b)
                wait_s(a, rows_a, ssem_a)

                @pl.when(p < SBE // BLK // 2 - 1)
                def _():
                    issue_g(a + 2, rows_a, gsem_a)

                issue_s(b, rows_b, ssem_b)

            wait_s(SBE // BLK - 1, rows_b, ssem_b)

        plsc.subcore_barrier()

        for k in range(ROWCHUNKS):
            base = (t * ROWCHUNKS + k) * CHUNK
            pltpu.sync_copy(s_sh.at[pl.ds(base, CHUNK)],
                            rows_a.at[pl.ds(0, CHUNK)])
            pltpu.sync_copy(a_sh.at[pl.ds(base, CHUNK)],
                            rows_a.at[pl.ds(CHUNK, CHUNK)])

            @plsc.parallel_loop(0, CHUNK, 1, unroll=2)
            def _(r):
                for q in range(4):
                    sl = pl.ds(q * 16, 16)
                    rows_a[r, sl] = _gelu_erf(rows_a[r, sl]
                                              + rows_a[CHUNK + r, sl])

            pltpu.sync_copy(rows_a.at[pl.ds(0, CHUNK)],
                            s_sh.at[pl.ds(base, CHUNK)])
        plsc.subcore_barrier()
        return carry

    lax.fori_loop(0, PROP_STEPS, step_body, 0)

    orows = OUTPUT_SIZE // NS
    pltpu.sync_copy(s_sh.at[pl.ds(INPUT_SIZE + t * orows, orows)],
                    out_hbm.at[pl.ds(t * orows, orows)])


@jax.jit
def kernel(x, weights, edge_index):
    row = edge_index[0]
    col = edge_index[1]
    pad = E_PAD - N_EDGES
    pad_idx = (jnp.arange(pad, dtype=jnp.int32) % N_NEURONS)
    col_p = jnp.concatenate([col, pad_idx]).reshape(NS, NSB, SBE)
    row_p = jnp.concatenate([row, pad_idx]).reshape(NS, NSB, SBE)
    w_p = jnp.concatenate(
        [weights, jnp.zeros((pad,), jnp.float32)]).reshape(NS, NSB, SBE)
    xt = x.T

    mesh = plsc.VectorSubcoreMesh(core_axis_name="c", subcore_axis_name="s",
                                  num_cores=1, num_subcores=NS)
    run = pl.kernel(
        _sc_body,
        out_type=jax.ShapeDtypeStruct((OUTPUT_SIZE, BATCH), jnp.float32),
        mesh=mesh,
        compiler_params=pltpu.CompilerParams(use_tc_tiling_on_sc=False),
        scratch_types=[
            pltpu.VMEM_SHARED((N_PAD, BATCH), jnp.float32),
            pltpu.VMEM_SHARED((N_PAD, BATCH), jnp.float32),
            pltpu.VMEM((SBE,), jnp.int32),
            pltpu.VMEM((SBE,), jnp.int32),
            pltpu.VMEM((SBE,), jnp.float32),
            pltpu.VMEM((BLK, BATCH), jnp.float32),
            pltpu.VMEM((BLK, BATCH), jnp.float32),
            pltpu.SemaphoreType.DMA,
            pltpu.SemaphoreType.DMA,
            pltpu.SemaphoreType.DMA,
            pltpu.SemaphoreType.DMA,
        ],
    )
    zeros_blk = jnp.zeros((CHUNK, BATCH), jnp.float32)
    out = run(xt, col_p, row_p, w_p, zeros_blk)
    return out.T


# speedup vs baseline: 1.5817x; 1.5391x over previous
import jax
import jax.numpy as jnp
from jax import lax
from jax.experimental import pallas as pl
from jax.experimental.pallas import tpu as pltpu
from jax.experimental.pallas import tpu_sc as plsc

N_NEURONS = 10000
N_EDGES = 500000
INPUT_SIZE = 512
OUTPUT_SIZE = 128
BATCH = 64
PROP_STEPS = 5

NC = 2
NS = 16
BLK = 256
SBE = 2048
NSB = 8
E_PAD = NC * NS * NSB * SBE
N_PAD = 10240
CHUNK = 128
ROWCHUNKS = N_PAD // (NS * CHUNK)


def _gelu_erf(v):
    z = v * 0.7071067811865476
    az = jnp.abs(z)
    t = 1.0 / (1.0 + 0.3275911 * az)
    poly = t * (0.254829592 + t * (-0.284496736 + t * (1.421413741
           + t * (-1.453152027 + t * 1.061405429))))
    erf_abs = 1.0 - poly * jnp.exp(-az * az)
    erf = jnp.where(z < 0.0, -erf_abs, erf_abs)
    return 0.5 * v * (1.0 + erf)


def _edge_phase(c, t, col_hbm, row_hbm, w_hbm, s_sh, a_sh,
                col_s, rid_s, w_s, rows_a, rows_b,
                gsem_a, gsem_b, ssem_a, ssem_b):
    def col_at(k):
        return col_s.at[pl.ds(k * BLK, BLK)]

    def rid_at(k):
        return rid_s.at[pl.ds(k * BLK, BLK)]

    def issue_g(k, rows, sem):
        pltpu.async_copy(s_sh.at[col_at(k)], rows, sem)

    def wait_g(k, rows, sem):
        pltpu.make_async_copy(s_sh.at[col_at(k)], rows, sem).wait()

    def issue_s(k, rows, sem):
        pltpu.async_copy(rows, a_sh.at[rid_at(k)], sem, add=True)

    def wait_s(k, rows, sem):
        pltpu.make_async_copy(rows, a_sh.at[rid_at(k)], sem).wait()

    def scale(k, rows):
        @plsc.parallel_loop(0, BLK // 16, 1, unroll=2)
        def _(g):
            wv = w_s[pl.ds(k * BLK + g * 16, 16)]
            for j in range(16):
                e = g * 16 + j
                w = wv[j]
                for q in range(4):
                    sl = pl.ds(q * 16, 16)
                    rows[e, sl] = w * rows[e, sl]

    @pl.loop(0, NSB)
    def _(sb):
        pltpu.sync_copy(col_hbm.at[c, t, sb], col_s)
        pltpu.sync_copy(row_hbm.at[c, t, sb], rid_s)
        pltpu.sync_copy(w_hbm.at[c, t, sb], w_s)
        issue_g(0, rows_a, gsem_a)

        @pl.loop(0, SBE // BLK // 2)
        def _(p):
            a = 2 * p
            b = 2 * p + 1
            wait_g(a, rows_a, gsem_a)

            @pl.when(p > 0)
            def _():
                wait_s(b - 2, rows_b, ssem_b)

            issue_g(b, rows_b, gsem_b)
            scale(a, rows_a)
            issue_s(a, rows_a, ssem_a)
            wait_g(b, rows_b, gsem_b)
            scale(b, rows_b)
            wait_s(a, rows_a, ssem_a)

            @pl.when(p < SBE // BLK // 2 - 1)
            def _():
                issue_g(a + 2, rows_a, gsem_a)

            issue_s(b, rows_b, ssem_b)

        wait_s(SBE // BLK - 1, rows_b, ssem_b)


def _zero_a(t, zeros_hbm, a_sh):
    for k in range(ROWCHUNKS):
        pltpu.sync_copy(zeros_hbm,
                        a_sh.at[pl.ds((t * ROWCHUNKS + k) * CHUNK, CHUNK)])


def _write_partial(c, t, a_sh, p0_hbm, p1_hbm):
    for k in range(ROWCHUNKS):
        base = (t * ROWCHUNKS + k) * CHUNK
        sl = pl.ds(base, CHUNK)

        @pl.when(c == 0)
        def _():
            pltpu.sync_copy(a_sh.at[sl], p0_hbm.at[sl])

        @pl.when(c == 1)
        def _():
            pltpu.sync_copy(a_sh.at[sl], p1_hbm.at[sl])


def _edge_body(s_in, col_hbm, row_hbm, w_hbm, zeros_hbm, p0_out, p1_out,
               s_sh, a_sh, col_s, rid_s, w_s, rows_a, rows_b,
               gsem_a, gsem_b, ssem_a, ssem_b):
    c = lax.axis_index("c")
    t = lax.axis_index("s")
    for k in range(ROWCHUNKS):
        sl = pl.ds((t * ROWCHUNKS + k) * CHUNK, CHUNK)
        pltpu.sync_copy(s_in.at[sl], s_sh.at[sl])
    _zero_a(t, zeros_hbm, a_sh)
    plsc.subcore_barrier()
    _edge_phase(c, t, col_hbm, row_hbm, w_hbm, s_sh, a_sh,
                col_s, rid_s, w_s, rows_a, rows_b,
                gsem_a, gsem_b, ssem_a, ssem_b)
    plsc.subcore_barrier()
    _write_partial(c, t, a_sh, p0_out, p1_out)


def _combine_edge_body(s_in, p0_in, p1_in, col_hbm, row_hbm, w_hbm, zeros_hbm,
                       s_out, p0_out, p1_out,
                       s_sh, a_sh, col_s, rid_s, w_s, rows_a, rows_b,
                       gsem_a, gsem_b, ssem_a, ssem_b):
    c = lax.axis_index("c")
    t = lax.axis_index("s")
    for k in range(ROWCHUNKS):
        base = (t * ROWCHUNKS + k) * CHUNK
        sl = pl.ds(base, CHUNK)
        pltpu.sync_copy(s_in.at[sl], rows_a.at[pl.ds(0, CHUNK)])
        pltpu.sync_copy(p0_in.at[sl], rows_a.at[pl.ds(CHUNK, CHUNK)])
        pltpu.sync_copy(p1_in.at[sl], rows_b.at[pl.ds(0, CHUNK)])

        @plsc.parallel_loop(0, CHUNK, 1, unroll=2)
        def _(r):
            for q in range(4):
                qs = pl.ds(q * 16, 16)
                rows_a[r, qs] = _gelu_erf(
                    rows_a[r, qs] + rows_a[CHUNK + r, qs] + rows_b[r, qs])

        pltpu.sync_copy(rows_a.at[pl.ds(0, CHUNK)], s_sh.at[sl])

        @pl.when(c == 0)
        def _():
            pltpu.sync_copy(rows_a.at[pl.ds(0, CHUNK)], s_out.at[sl])

    _zero_a(t, zeros_hbm, a_sh)
    plsc.subcore_barrier()
    _edge_phase(c, t, col_hbm, row_hbm, w_hbm, s_sh, a_sh,
                col_s, rid_s, w_s, rows_a, rows_b,
                gsem_a, gsem_b, ssem_a, ssem_b)
    plsc.subcore_barrier()
    _write_partial(c, t, a_sh, p0_out, p1_out)


def _final_body(s_in, p0_in, p1_in, out_hbm, buf_s, buf_p0, buf_p1):
    c = lax.axis_index("c")
    t = lax.axis_index("s")
    orows = OUTPUT_SIZE // NS

    @pl.when(c == 0)
    def _():
        base = INPUT_SIZE + t * orows
        pltpu.sync_copy(s_in.at[pl.ds(base, orows)], buf_s)
        pltpu.sync_copy(p0_in.at[pl.ds(base, orows)], buf_p0)
        pltpu.sync_copy(p1_in.at[pl.ds(base, orows)], buf_p1)

        @pl.loop(0, orows)
        def _(r):
            for q in range(4):
                qs = pl.ds(q * 16, 16)
                buf_s[r, qs] = _gelu_erf(
                    buf_s[r, qs] + buf_p0[r, qs] + buf_p1[r, qs])

        pltpu.sync_copy(buf_s, out_hbm.at[pl.ds(t * orows, orows)])


import functools


@functools.cache
def _build_kernels():
    mesh = plsc.VectorSubcoreMesh(core_axis_name="c", subcore_axis_name="s",
                                  num_cores=NC, num_subcores=NS)
    cparams = pltpu.CompilerParams(use_tc_tiling_on_sc=False)
    sb = jax.ShapeDtypeStruct((N_PAD, BATCH), jnp.float32)
    edge_scratch = [
        pltpu.VMEM_SHARED((N_PAD, BATCH), jnp.float32),
        pltpu.VMEM_SHARED((N_PAD, BATCH), jnp.float32),
        pltpu.VMEM((SBE,), jnp.int32),
        pltpu.VMEM((SBE,), jnp.int32),
        pltpu.VMEM((SBE,), jnp.float32),
        pltpu.VMEM((BLK, BATCH), jnp.float32),
        pltpu.VMEM((BLK, BATCH), jnp.float32),
        pltpu.SemaphoreType.DMA,
        pltpu.SemaphoreType.DMA,
        pltpu.SemaphoreType.DMA,
        pltpu.SemaphoreType.DMA,
    ]
    k_edge = pl.kernel(
        _edge_body, out_type=[sb, sb], mesh=mesh, compiler_params=cparams,
        scratch_types=list(edge_scratch))
    k_combine_edge = pl.kernel(
        _combine_edge_body, out_type=[sb, sb, sb], mesh=mesh,
        compiler_params=cparams, scratch_types=list(edge_scratch))
    k_final = pl.kernel(
        _final_body,
        out_type=jax.ShapeDtypeStruct((OUTPUT_SIZE, BATCH), jnp.float32),
        mesh=mesh, compiler_params=cparams,
        scratch_types=[
            pltpu.VMEM((OUTPUT_SIZE // NS, BATCH), jnp.float32),
            pltpu.VMEM((OUTPUT_SIZE // NS, BATCH), jnp.float32),
            pltpu.VMEM((OUTPUT_SIZE // NS, BATCH), jnp.float32),
        ])
    return k_edge, k_combine_edge, k_final


@jax.jit
def kernel(x, weights, edge_index):
    row = edge_index[0]
    col = edge_index[1]
    pad = E_PAD - N_EDGES
    pad_idx = (jnp.arange(pad, dtype=jnp.int32) % N_NEURONS)
    col_p = jnp.concatenate([col, pad_idx]).reshape(NC, NS, NSB, SBE)
    row_p = jnp.concatenate([row, pad_idx]).reshape(NC, NS, NSB, SBE)
    w_p = jnp.concatenate(
        [weights, jnp.zeros((pad,), jnp.float32)]).reshape(NC, NS, NSB, SBE)
    s0 = jnp.zeros((N_PAD, BATCH), jnp.float32).at[:INPUT_SIZE].set(x.T)
    zeros_blk = jnp.zeros((CHUNK, BATCH), jnp.float32)

    k_edge, k_combine_edge, k_final = _build_kernels()
    p0, p1 = k_edge(s0, col_p, row_p, w_p, zeros_blk)
    s = s0
    for _ in range(PROP_STEPS - 1):
        s, p0, p1 = k_combine_edge(s, p0, p1, col_p, row_p, w_p, zeros_blk)
    out = k_final(s, p0, p1)
    return out.T


# speedup vs baseline: 1.7185x; 1.0865x over previous
import jax
import jax.numpy as jnp
from jax import lax
from jax.experimental import pallas as pl
from jax.experimental.pallas import tpu as pltpu
from jax.experimental.pallas import tpu_sc as plsc

N_NEURONS = 10000
N_EDGES = 500000
INPUT_SIZE = 512
OUTPUT_SIZE = 128
BATCH = 64
PROP_STEPS = 5

NC = 2
NS = 16
BLK = 256
SBE = 2048
NSB = 8
E_PAD = NC * NS * NSB * SBE
N_PAD = 10240
CHUNK = 128
ROWCHUNKS = N_PAD // (NS * CHUNK)


def _gelu_erf(v):
    z = v * 0.7071067811865476
    az = jnp.abs(z)
    t = 1.0 / (1.0 + 0.3275911 * az)
    poly = t * (0.254829592 + t * (-0.284496736 + t * (1.421413741
           + t * (-1.453152027 + t * 1.061405429))))
    erf_abs = 1.0 - poly * jnp.exp(-az * az)
    erf = jnp.where(z < 0.0, -erf_abs, erf_abs)
    return 0.5 * v * (1.0 + erf)


def _edge_phase(c, t, col_hbm, row_hbm, w_hbm, s_sh, a_sh,
                col_s, rid_s, w_s, rows_a, rows_b,
                gsem_a, gsem_b, ssem_a, ssem_b, stg_sem):
    def col_at(k):
        return col_s.at[pl.ds(k * BLK, BLK)]

    def rid_at(k):
        return rid_s.at[pl.ds(k * BLK, BLK)]

    def issue_g(k, rows, sem):
        pltpu.async_copy(s_sh.at[col_at(k)], rows, sem)

    def wait_g(k, rows, sem):
        pltpu.make_async_copy(s_sh.at[col_at(k)], rows, sem).wait()

    def issue_s(k, rows, sem):
        pltpu.async_copy(rows, a_sh.at[rid_at(k)], sem, add=True)

    def wait_s(k, rows, sem):
        pltpu.make_async_copy(rows, a_sh.at[rid_at(k)], sem).wait()

    def scale(k, rows):
        @plsc.parallel_loop(0, BLK // 16, 1, unroll=2)
        def _(g):
            wv = w_s[pl.ds(k * BLK + g * 16, 16)]
            for j in range(16):
                e = g * 16 + j
                w = wv[j]
                for q in range(4):
                    sl = pl.ds(q * 16, 16)
                    rows[e, sl] = w * rows[e, sl]

    @pl.loop(0, NSB)
    def _(sb):
        d1 = pltpu.async_copy(col_hbm.at[c, t, sb], col_s, stg_sem)
        d2 = pltpu.async_copy(row_hbm.at[c, t, sb], rid_s, stg_sem)
        d3 = pltpu.async_copy(w_hbm.at[c, t, sb], w_s, stg_sem)
        d1.wait()
        d2.wait()
        d3.wait()
        issue_g(0, rows_a, gsem_a)

        @pl.loop(0, SBE // BLK // 2)
        def _(p):
            a = 2 * p
            b = 2 * p + 1
            wait_g(a, rows_a, gsem_a)

            @pl.when(p > 0)
            def _():
                wait_s(b - 2, rows_b, ssem_b)

            issue_g(b, rows_b, gsem_b)
            scale(a, rows_a)
            issue_s(a, rows_a, ssem_a)
            wait_g(b, rows_b, gsem_b)
            scale(b, rows_b)
            wait_s(a, rows_a, ssem_a)

            @pl.when(p < SBE // BLK // 2 - 1)
            def _():
                issue_g(a + 2, rows_a, gsem_a)

            issue_s(b, rows_b, ssem_b)

        wait_s(SBE // BLK - 1, rows_b, ssem_b)


def _zero_a(t, zeros_hbm, a_sh, sem):
    ds = [pltpu.async_copy(
        zeros_hbm, a_sh.at[pl.ds((t * ROWCHUNKS + k) * CHUNK, CHUNK)], sem)
        for k in range(ROWCHUNKS)]
    for d in ds:
        d.wait()


def _write_partial(c, t, a_sh, p0_hbm, p1_hbm, sem):
    @pl.when(c == 0)
    def _():
        ds = [pltpu.async_copy(
            a_sh.at[pl.ds((t * ROWCHUNKS + k) * CHUNK, CHUNK)],
            p0_hbm.at[pl.ds((t * ROWCHUNKS + k) * CHUNK, CHUNK)], sem)
            for k in range(ROWCHUNKS)]
        for d in ds:
            d.wait()

    @pl.when(c == 1)
    def _():
        ds = [pltpu.async_copy(
            a_sh.at[pl.ds((t * ROWCHUNKS + k) * CHUNK, CHUNK)],
            p1_hbm.at[pl.ds((t * ROWCHUNKS + k) * CHUNK, CHUNK)], sem)
            for k in range(ROWCHUNKS)]
        for d in ds:
            d.wait()


def _edge_body(s_in, col_hbm, row_hbm, w_hbm, zeros_hbm, p0_out, p1_out,
               s_sh, a_sh, col_s, rid_s, w_s, rows_a, rows_b,
               gsem_a, gsem_b, ssem_a, ssem_b, stg_sem):
    c = lax.axis_index("c")
    t = lax.axis_index("s")
    ds = [pltpu.async_copy(
        s_in.at[pl.ds((t * ROWCHUNKS + k) * CHUNK, CHUNK)],
        s_sh.at[pl.ds((t * ROWCHUNKS + k) * CHUNK, CHUNK)], stg_sem)
        for k in range(ROWCHUNKS)]
    for d in ds:
        d.wait()
    _zero_a(t, zeros_hbm, a_sh, stg_sem)
    plsc.subcore_barrier()
    _edge_phase(c, t, col_hbm, row_hbm, w_hbm, s_sh, a_sh,
                col_s, rid_s, w_s, rows_a, rows_b,
                gsem_a, gsem_b, ssem_a, ssem_b, stg_sem)
    plsc.subcore_barrier()
    _write_partial(c, t, a_sh, p0_out, p1_out, stg_sem)


def _combine_edge_body(s_in, p0_in, p1_in, col_hbm, row_hbm, w_hbm, zeros_hbm,
                       s_out, p0_out, p1_out,
                       s_sh, a_sh, col_s, rid_s, w_s, rows_a, rows_b,
                       gsem_a, gsem_b, ssem_a, ssem_b, stg_sem):
    c = lax.axis_index("c")
    t = lax.axis_index("s")
    for k in range(ROWCHUNKS):
        base = (t * ROWCHUNKS + k) * CHUNK
        sl = pl.ds(base, CHUNK)
        d1 = pltpu.async_copy(s_in.at[sl], rows_a.at[pl.ds(0, CHUNK)],
                              stg_sem)
        d2 = pltpu.async_copy(p0_in.at[sl], rows_a.at[pl.ds(CHUNK, CHUNK)],
                              stg_sem)
        d3 = pltpu.async_copy(p1_in.at[sl], rows_b.at[pl.ds(0, CHUNK)],
                              stg_sem)
        d1.wait()
        d2.wait()
        d3.wait()

        @plsc.parallel_loop(0, CHUNK, 1, unroll=2)
        def _(r):
            for q in range(4):
                qs = pl.ds(q * 16, 16)
                rows_a[r, qs] = _gelu_erf(
                    rows_a[r, qs] + rows_a[CHUNK + r, qs] + rows_b[r, qs])

        pltpu.sync_copy(rows_a.at[pl.ds(0, CHUNK)], s_sh.at[sl])

        @pl.when(c == 0)
        def _():
            pltpu.sync_copy(rows_a.at[pl.ds(0, CHUNK)], s_out.at[sl])

    _zero_a(t, zeros_hbm, a_sh, stg_sem)
    plsc.subcore_barrier()
    _edge_phase(c, t, col_hbm, row_hbm, w_hbm, s_sh, a_sh,
                col_s, rid_s, w_s, rows_a, rows_b,
                gsem_a, gsem_b, ssem_a, ssem_b, stg_sem)
    plsc.subcore_barrier()
    _write_partial(c, t, a_sh, p0_out, p1_out, stg_sem)


def _final_body(s_in, p0_in, p1_in, out_hbm, buf_s, buf_p0, buf_p1):
    c = lax.axis_index("c")
    t = lax.axis_index("s")
    orows = OUTPUT_SIZE // NS

    @pl.when(c == 0)
    def _():
        base = INPUT_SIZE + t * orows
        pltpu.sync_copy(s_in.at[pl.ds(base, orows)], buf_s)
        pltpu.sync_copy(p0_in.at[pl.ds(base, orows)], buf_p0)
        pltpu.sync_copy(p1_in.at[pl.ds(base, orows)], buf_p1)

        @pl.loop(0, orows)
        def _(r):
            for q in range(4):
                qs = pl.ds(q * 16, 16)
                buf_s[r, qs] = _gelu_erf(
                    buf_s[r, qs] + buf_p0[r, qs] + buf_p1[r, qs])

        pltpu.sync_copy(buf_s, out_hbm.at[pl.ds(t * orows, orows)])


import functools


@functools.cache
def _build_kernels():
    mesh = plsc.VectorSubcoreMesh(core_axis_name="c", subcore_axis_name="s",
                                  num_cores=NC, num_subcores=NS)
    cparams = pltpu.CompilerParams(use_tc_tiling_on_sc=False)
    sb = jax.ShapeDtypeStruct((N_PAD, BATCH), jnp.float32)
    edge_scratch = [
        pltpu.VMEM_SHARED((N_PAD, BATCH), jnp.float32),
        pltpu.VMEM_SHARED((N_PAD, BATCH), jnp.float32),
        pltpu.VMEM((SBE,), jnp.int32),
        pltpu.VMEM((SBE,), jnp.int32),
        pltpu.VMEM((SBE,), jnp.float32),
        pltpu.VMEM((BLK, BATCH), jnp.float32),
        pltpu.VMEM((BLK, BATCH), jnp.float32),
        pltpu.SemaphoreType.DMA,
        pltpu.SemaphoreType.DMA,
        pltpu.SemaphoreType.DMA,
        pltpu.SemaphoreType.DMA,
        pltpu.SemaphoreType.DMA,
    ]
    k_edge = pl.kernel(
        _edge_body, out_type=[sb, sb], mesh=mesh, compiler_params=cparams,
        scratch_types=list(edge_scratch))
    k_combine_edge = pl.kernel(
        _combine_edge_body, out_type=[sb, sb, sb], mesh=mesh,
        compiler_params=cparams, scratch_types=list(edge_scratch))
    k_final = pl.kernel(
        _final_body,
        out_type=jax.ShapeDtypeStruct((OUTPUT_SIZE, BATCH), jnp.float32),
        mesh=mesh, compiler_params=cparams,
        scratch_types=[
            pltpu.VMEM((OUTPUT_SIZE // NS, BATCH), jnp.float32),
            pltpu.VMEM((OUTPUT_SIZE // NS, BATCH), jnp.float32),
            pltpu.VMEM((OUTPUT_SIZE // NS, BATCH), jnp.float32),
        ])
    return k_edge, k_combine_edge, k_final


@jax.jit
def kernel(x, weights, edge_index):
    row = edge_index[0]
    col = edge_index[1]
    pad = E_PAD - N_EDGES
    pad_idx = (jnp.arange(pad, dtype=jnp.int32) % N_NEURONS)
    col_p = jnp.concatenate([col, pad_idx]).reshape(NC, NS, NSB, SBE)
    row_p = jnp.concatenate([row, pad_idx]).reshape(NC, NS, NSB, SBE)
    w_p = jnp.concatenate(
        [weights, jnp.zeros((pad,), jnp.float32)]).reshape(NC, NS, NSB, SBE)
    s0 = jnp.zeros((N_PAD, BATCH), jnp.float32).at[:INPUT_SIZE].set(x.T)
    zeros_blk = jnp.zeros((CHUNK, BATCH), jnp.float32)

    k_edge, k_combine_edge, k_final = _build_kernels()
    p0, p1 = k_edge(s0, col_p, row_p, w_p, zeros_blk)
    s = s0
    for _ in range(PROP_STEPS - 1):
        s, p0, p1 = k_combine_edge(s, p0, p1, col_p, row_p, w_p, zeros_blk)
    out = k_final(s, p0, p1)
    return out.T
